# Initial kernel scaffold; baseline (speedup 1.0000x reference)
import jax
import jax.numpy as jnp
from jax.experimental import pallas as pl


def kernel(output, mask, ind, target):
    raise NotImplementedError("write your pallas kernel here")


# speedup vs baseline: 1.7374x; 1.7374x over previous
import jax
import jax.numpy as jnp
from jax import lax
from jax.experimental import pallas as pl
from jax.experimental.pallas import tpu as pltpu, tpu_sc as plsc

B, C, H, W = 64, 2, 256, 256
K = 128
HWP = H * W
CHW = C * H * W
NW = 16
BPW = B // NW
EPW = BPW * K * C
NCHUNK = EPW // 16
NROW = EPW // 128


def _body(out_hbm, ind_hbm, mask_hbm, targ_hbm, part_hbm, res_hbm,
          ind_v, mask_v, targ_v, idx_v, pred_v, part_v, all_v, res_v, sem):
    w = lax.axis_index("s")

    pltpu.sync_copy(ind_hbm.at[pl.ds(w * BPW * K, BPW * K)], ind_v)
    pltpu.sync_copy(mask_hbm.at[pl.ds(w * BPW * K, BPW * K)], mask_v)
    pltpu.sync_copy(targ_hbm.at[pl.ds(w * EPW, EPW)], targ_v)

    for j in range(NCHUNK):
        c = j >> 5
        bl = (j >> 3) & 3
        kb = (j & 7) * 16
        indc = ind_v[pl.ds(bl * K + kb, 16)]
        base = (BPW * w + bl) * CHW + c * HWP
        idx_v[j >> 3, pl.ds((j & 7) * 16, 16)] = indc + base

    descs = [pltpu.async_copy(out_hbm.at[idx_v.at[r]], pred_v.at[r], sem)
             for r in range(NROW)]
    for d in descs:
        d.wait()

    acc = jnp.zeros((16,), jnp.float32)
    for j in range(NCHUNK):
        c = j >> 5
        bl = (j >> 3) & 3
        kb = (j & 7) * 16
        pred = pred_v[j >> 3, pl.ds((j & 7) * 16, 16)]
        m = mask_v[pl.ds(bl * K + kb, 16)].astype(jnp.float32)
        t = targ_v[pl.ds(bl * K * C + c * K + kb, 16)]
        acc = acc + jnp.abs(pred * m - t * m)

    part_v[...] = acc
    pltpu.sync_copy(part_v, part_hbm.at[w])
    plsc.subcore_barrier()

    @pl.when(w == 0)
    def _():
        pltpu.sync_copy(part_hbm, all_v)
        tot = all_v[0]
        for r in range(1, NW):
            tot = tot + all_v[r]
        s = tot[0]
        for l in range(1, 16):
            s = s + tot[l]
        res_v[...] = jnp.broadcast_to(s * (1.0 / (B * K * C)), (16,))
        pltpu.sync_copy(res_v, res_hbm)


_l1 = pl.kernel(
    _body,
    out_type=(
        jax.ShapeDtypeStruct((NW, 16), jnp.float32),
        jax.ShapeDtypeStruct((16,), jnp.float32),
    ),
    mesh=plsc.VectorSubcoreMesh(core_axis_name="c", subcore_axis_name="s",
                                num_cores=1),
    scratch_types=[
        pltpu.VMEM((BPW * K,), jnp.int32),
        pltpu.VMEM((BPW * K,), jnp.int32),
        pltpu.VMEM((EPW,), jnp.float32),
        pltpu.VMEM((NROW, 128), jnp.int32),
        pltpu.VMEM((NROW, 128), jnp.float32),
        pltpu.VMEM((16,), jnp.float32),
        pltpu.VMEM((NW, 16), jnp.float32),
        pltpu.VMEM((16,), jnp.float32),
        pltpu.SemaphoreType.DMA,
    ],
)


def kernel(output, mask, ind, target):
    out_flat = output.reshape(-1)
    ind_flat = ind.astype(jnp.int32).reshape(-1)
    mask_flat = mask.astype(jnp.int32).reshape(-1)
    targ_flat = jnp.transpose(target, (0, 2, 1)).reshape(-1)
    _, res = _l1(out_flat, ind_flat, mask_flat, targ_flat)
    return res[0]


# speedup vs baseline: 2.4390x; 1.4038x over previous
import jax
import jax.numpy as jnp
from jax import lax
from jax.experimental import pallas as pl
from jax.experimental.pallas import tpu as pltpu, tpu_sc as plsc

B, C, H, W = 64, 2, 256, 256
NROWS = B * C * H
K = 128
NW = 16
BPW = B // NW
EPW = BPW * K * C
NCH = 8


def _body(out_hbm, ind_hbm, mask_hbm, targ_hbm, part_hbm, res_hbm,
          ind_v, mask_v, targ_v, ridx_v, rows_v, part_v, all_v, res_v,
          sem0, sem1):
    w = lax.axis_index("s")
    out2 = out_hbm.reshape(NROWS, W)
    sems = [sem0, sem1]

    pltpu.sync_copy(ind_hbm.at[pl.ds(w * BPW * K, BPW * K)], ind_v)
    pltpu.sync_copy(mask_hbm.at[pl.ds(w * BPW * K, BPW * K)], mask_v)
    pltpu.sync_copy(targ_hbm.at[pl.ds(w * EPW, EPW)], targ_v)

    iota16 = lax.iota(jnp.int32, 16)

    for j in range(NCH):
        c = j >> 2
        bl = j & 3
        rowbase = (BPW * w + bl) * C * H + c * H
        for g in range(8):
            ind16 = ind_v[pl.ds(bl * K + g * 16, 16)]
            ridx_v[j, pl.ds(g * 16, 16)] = (
                lax.shift_right_logical(ind16, 8) + rowbase)

    def fire(j):
        return pltpu.async_copy(out2.at[ridx_v.at[j]], rows_v.at[j & 1],
                                sems[j & 1])

    acc = jnp.zeros((16,), jnp.float32)
    d = fire(0)
    for j in range(NCH):
        d_next = fire(j + 1) if j + 1 < NCH else None
        d.wait()
        c = j >> 2
        bl = j & 3
        buf = rows_v.at[j & 1]
        for g in range(8):
            ind16 = ind_v[pl.ds(bl * K + g * 16, 16)]
            col = ind16 & 255
            pred = plsc.load_gather(buf, [iota16 + g * 16, col])
            m = mask_v[pl.ds(bl * K + g * 16, 16)].astype(jnp.float32)
            t = targ_v[pl.ds(bl * K * C + c * K + g * 16, 16)]
            acc = acc + jnp.abs(pred * m - t * m)
        d = d_next

    part_v[...] = acc
    pltpu.sync_copy(part_v, part_hbm.at[w])
    plsc.subcore_barrier()

    @pl.when(w == 0)
    def _():
        pltpu.sync_copy(part_hbm, all_v)
        tot = all_v[0]
        for r in range(1, NW):
            tot = tot + all_v[r]
        s = tot[0]
        for l in range(1, 16):
            s = s + tot[l]
        res_v[...] = jnp.broadcast_to(s * (1.0 / (B * K * C)), (16,))
        pltpu.sync_copy(res_v, res_hbm)


_l1 = pl.kernel(
    _body,
    out_type=(
        jax.ShapeDtypeStruct((NW, 16), jnp.float32),
        jax.ShapeDtypeStruct((16,), jnp.float32),
    ),
    mesh=plsc.VectorSubcoreMesh(core_axis_name="c", subcore_axis_name="s",
                                num_cores=1),
    compiler_params=pltpu.CompilerParams(use_tc_tiling_on_sc=True,
                                         needs_layout_passes=False),
    scratch_types=[
        pltpu.VMEM((BPW * K,), jnp.int32),
        pltpu.VMEM((BPW * K,), jnp.int32),
        pltpu.VMEM((EPW,), jnp.float32),
        pltpu.VMEM((NCH, 128), jnp.int32),
        pltpu.VMEM((2, 128, W), jnp.float32),
        pltpu.VMEM((16,), jnp.float32),
        pltpu.VMEM((NW, 16), jnp.float32),
        pltpu.VMEM((16,), jnp.float32),
        pltpu.SemaphoreType.DMA,
        pltpu.SemaphoreType.DMA,
    ],
)


def kernel(output, mask, ind, target):
    ind_flat = ind.astype(jnp.int32).reshape(-1)
    mask_flat = mask.astype(jnp.int32).reshape(-1)
    targ_flat = jnp.transpose(target, (0, 2, 1)).reshape(-1)
    _, res = _l1(output, ind_flat, mask_flat, targ_flat)
    return res[0]


# speedup vs baseline: 2.6686x; 1.0941x over previous
import jax
import jax.numpy as jnp
from jax import lax
from jax.experimental import pallas as pl
from jax.experimental.pallas import tpu as pltpu, tpu_sc as plsc

B, C, H, W = 64, 2, 256, 256
NROWS = B * C * H
K = 128
NC = 2
NSC = 16
NWT = NC * NSC
BPW = B // NWT
EPW = BPW * K * C
NCH = EPW // 128


def _body(out_hbm, ind_hbm, mask_hbm, targ_hbm, part_hbm, res_hbm,
          ind_v, mask_v, targ_v, ridx_v, rows_v, part_v, all_v, res_v,
          sem0, sem1):
    cid = lax.axis_index("c")
    sid = lax.axis_index("s")
    wid = sid * NC + cid
    out2 = out_hbm.reshape(NROWS, W)
    sems = [sem0, sem1]

    pltpu.sync_copy(ind_hbm.at[pl.ds(wid * BPW * K, BPW * K)], ind_v)
    pltpu.sync_copy(mask_hbm.at[pl.ds(wid * BPW * K, BPW * K)], mask_v)
    pltpu.sync_copy(targ_hbm.at[pl.ds(wid * EPW, EPW)], targ_v)

    iota16 = lax.iota(jnp.int32, 16)

    for j in range(NCH):
        c = j >> 1
        bl = j & 1
        rowbase = ((BPW * wid + bl) * C + c) * H
        for g in range(8):
            ind16 = ind_v[pl.ds(bl * K + g * 16, 16)]
            ridx_v[j, pl.ds(g * 16, 16)] = (
                lax.shift_right_logical(ind16, 8) + rowbase)

    def fire(j):
        return pltpu.async_copy(out2.at[ridx_v.at[j]], rows_v.at[j & 1],
                                sems[j & 1])

    acc = jnp.zeros((16,), jnp.float32)
    d = fire(0)
    for j in range(NCH):
        d_next = fire(j + 1) if j + 1 < NCH else None
        d.wait()
        c = j >> 1
        bl = j & 1
        buf = rows_v.at[j & 1]
        for g in range(8):
            ind16 = ind_v[pl.ds(bl * K + g * 16, 16)]
            col = ind16 & 255
            pred = plsc.load_gather(buf, [iota16 + g * 16, col])
            m = mask_v[pl.ds(bl * K + g * 16, 16)].astype(jnp.float32)
            t = targ_v[pl.ds(bl * K * C + c * K + g * 16, 16)]
            acc = acc + jnp.abs(pred * m - t * m)
        d = d_next

    part_v[...] = acc
    pltpu.sync_copy(part_v, part_hbm.at[cid, sid])
    plsc.subcore_barrier()

    @pl.when(sid == 0)
    def _():
        pltpu.sync_copy(part_hbm.at[cid], all_v)
        tot = all_v[0]
        for r in range(1, NSC):
            tot = tot + all_v[r]
        s = tot[0]
        for l in range(1, 16):
            s = s + tot[l]
        res_v[...] = jnp.broadcast_to(s * (1.0 / (B * K * C)), (16,))
        pltpu.sync_copy(res_v, res_hbm.at[cid])


_l1 = pl.kernel(
    _body,
    out_type=(
        jax.ShapeDtypeStruct((NC, NSC, 16), jnp.float32),
        jax.ShapeDtypeStruct((NC, 16), jnp.float32),
    ),
    mesh=plsc.VectorSubcoreMesh(core_axis_name="c", subcore_axis_name="s",
                                num_cores=NC),
    compiler_params=pltpu.CompilerParams(use_tc_tiling_on_sc=True,
                                         needs_layout_passes=False),
    scratch_types=[
        pltpu.VMEM((BPW * K,), jnp.int32),
        pltpu.VMEM((BPW * K,), jnp.int32),
        pltpu.VMEM((EPW,), jnp.float32),
        pltpu.VMEM((NCH, 128), jnp.int32),
        pltpu.VMEM((2, 128, W), jnp.float32),
        pltpu.VMEM((16,), jnp.float32),
        pltpu.VMEM((NSC, 16), jnp.float32),
        pltpu.VMEM((16,), jnp.float32),
        pltpu.SemaphoreType.DMA,
        pltpu.SemaphoreType.DMA,
    ],
)


def kernel(output, mask, ind, target):
    ind_flat = ind.astype(jnp.int32).reshape(-1)
    mask_flat = mask.astype(jnp.int32).reshape(-1)
    targ_flat = jnp.transpose(target, (0, 2, 1)).reshape(-1)
    _, res = _l1(output, ind_flat, mask_flat, targ_flat)
    return res[0, 0] + res[1, 0]


# speedup vs baseline: 2.6942x; 1.0096x over previous
import jax
import jax.numpy as jnp
from jax import lax
from jax.experimental import pallas as pl
from jax.experimental.pallas import tpu as pltpu, tpu_sc as plsc

B, C, H, W = 64, 2, 256, 256
NROWS = B * C * H
K = 128
NC = 2
NSC = 16
NWT = NC * NSC
BPW = B // NWT
EPW = BPW * K * C
CHUNK = 64
NCH = EPW // CHUNK
NBUF = 4


def _body(out_hbm, ind_hbm, mask_hbm, targ_hbm, part_hbm, res_hbm,
          ind_v, mask_v, targ_v, ridx_v, rows_v, part_v, all_v, res_v,
          *sems):
    cid = lax.axis_index("c")
    sid = lax.axis_index("s")
    wid = sid * NC + cid
    out2 = out_hbm.reshape(NROWS, W)

    pltpu.sync_copy(ind_hbm.at[pl.ds(wid * BPW, BPW)], ind_v)
    pltpu.sync_copy(mask_hbm.at[pl.ds(wid * BPW, BPW)], mask_v)
    pltpu.sync_copy(targ_hbm.at[pl.ds(wid * EPW, EPW)], targ_v)

    iota16 = lax.iota(jnp.int32, 16)

    for j in range(NCH):
        c = j >> 2
        bl = (j >> 1) & 1
        kb = (j & 1) * CHUNK
        rowbase = ((BPW * wid + bl) * C + c) * H
        for g in range(4):
            ind16 = ind_v[bl, pl.ds(kb + g * 16, 16)]
            ridx_v[j, pl.ds(g * 16, 16)] = (
                lax.shift_right_logical(ind16, 8) + rowbase)

    def fire(j):
        return pltpu.async_copy(out2.at[ridx_v.at[j]], rows_v.at[j % NBUF],
                                sems[j % NBUF])

    descs = [fire(j) for j in range(NBUF)]
    acc = jnp.zeros((16,), jnp.float32)
    for j in range(NCH):
        descs[j % NBUF].wait()
        c = j >> 2
        bl = (j >> 1) & 1
        kb = (j & 1) * CHUNK
        buf = rows_v.at[j % NBUF]
        for g in range(4):
            ind16 = ind_v[bl, pl.ds(kb + g * 16, 16)]
            col = ind16 & 255
            pred = plsc.load_gather(buf, [iota16 + g * 16, col])
            m = mask_v[bl, pl.ds(kb + g * 16, 16)].astype(jnp.float32)
            t = targ_v[pl.ds(bl * K * C + c * K + kb + g * 16, 16)]
            acc = acc + jnp.abs(pred * m - t * m)
        if j + NBUF < NCH:
            descs[j % NBUF] = fire(j + NBUF)

    part_v[...] = acc
    pltpu.sync_copy(part_v, part_hbm.at[cid, sid])
    plsc.subcore_barrier()

    @pl.when(sid == 0)
    def _():
        pltpu.sync_copy(part_hbm.at[cid], all_v)
        tot = all_v[0]
        for r in range(1, NSC):
            tot = tot + all_v[r]
        s = tot[0]
        for l in range(1, 16):
            s = s + tot[l]
        res_v[...] = jnp.broadcast_to(s * (1.0 / (B * K * C)), (16,))
        pltpu.sync_copy(res_v, res_hbm.at[cid])


_l1 = pl.kernel(
    _body,
    out_type=(
        jax.ShapeDtypeStruct((NC, NSC, 16), jnp.float32),
        jax.ShapeDtypeStruct((NC, 16), jnp.float32),
    ),
    mesh=plsc.VectorSubcoreMesh(core_axis_name="c", subcore_axis_name="s",
                                num_cores=NC),
    compiler_params=pltpu.CompilerParams(use_tc_tiling_on_sc=True,
                                         needs_layout_passes=False),
    scratch_types=[
        pltpu.VMEM((BPW, K), jnp.int32),
        pltpu.VMEM((BPW, K), jnp.int32),
        pltpu.VMEM((EPW,), jnp.float32),
        pltpu.VMEM((NCH, CHUNK), jnp.int32),
        pltpu.VMEM((NBUF, CHUNK, W), jnp.float32),
        pltpu.VMEM((16,), jnp.float32),
        pltpu.VMEM((NSC, 16), jnp.float32),
        pltpu.VMEM((16,), jnp.float32),
    ] + [pltpu.SemaphoreType.DMA] * NBUF,
)


def kernel(output, mask, ind, target):
    ind32 = ind.astype(jnp.int32)
    targ_flat = jnp.transpose(target, (0, 2, 1)).reshape(-1)
    _, res = _l1(output, ind32, mask, targ_flat)
    return res[0, 0] + res[1, 0]


# speedup vs baseline: 2.7813x; 1.0323x over previous
import jax
import jax.numpy as jnp
from jax import lax
from jax.experimental import pallas as pl
from jax.experimental.pallas import tpu as pltpu, tpu_sc as plsc

B, C, H, W = 64, 2, 256, 256
NROWS = B * C * H
K = 128
NC = 2
NSC = 16
NWT = NC * NSC
BPW = B // NWT
EPW = BPW * K * C
CHUNK = 64
NCH = EPW // CHUNK
NBUF = 6


def _body(out_hbm, ind_hbm, mask_hbm, targ_hbm, part_hbm, res_hbm,
          ind_v, mask_v, targ_v, ridx_v, rows_v, part_v, all_v, res_v,
          *sems):
    cid = lax.axis_index("c")
    sid = lax.axis_index("s")
    wid = sid * NC + cid
    out2 = out_hbm.reshape(NROWS, W)

    pltpu.sync_copy(ind_hbm.at[pl.ds(wid * BPW, BPW)], ind_v)
    pltpu.sync_copy(mask_hbm.at[pl.ds(wid * BPW, BPW)], mask_v)
    pltpu.sync_copy(targ_hbm.at[pl.ds(wid * EPW, EPW)], targ_v)

    iota16 = lax.iota(jnp.int32, 16)

    for j in range(NCH):
        c = j >> 2
        bl = (j >> 1) & 1
        kb = (j & 1) * CHUNK
        rowbase = ((BPW * wid + bl) * C + c) * H
        for g in range(4):
            ind16 = ind_v[bl, pl.ds(kb + g * 16, 16)]
            ridx_v[j, pl.ds(g * 16, 16)] = (
                lax.shift_right_logical(ind16, 8) + rowbase)

    def fire(j):
        return pltpu.async_copy(out2.at[ridx_v.at[j]], rows_v.at[j % NBUF],
                                sems[j % NBUF])

    descs = [fire(j) for j in range(NBUF)]
    acc = jnp.zeros((16,), jnp.float32)
    for j in range(NCH):
        descs[j % NBUF].wait()
        c = j >> 2
        bl = (j >> 1) & 1
        kb = (j & 1) * CHUNK
        buf = rows_v.at[j % NBUF]
        for g in range(4):
            ind16 = ind_v[bl, pl.ds(kb + g * 16, 16)]
            col = ind16 & 255
            pred = plsc.load_gather(buf, [iota16 + g * 16, col])
            m = mask_v[bl, pl.ds(kb + g * 16, 16)].astype(jnp.float32)
            t = targ_v[pl.ds(bl * K * C + c * K + kb + g * 16, 16)]
            acc = acc + jnp.abs(pred * m - t * m)
        if j + NBUF < NCH:
            descs[j % NBUF] = fire(j + NBUF)

    part_v[...] = acc
    pltpu.sync_copy(part_v, part_hbm.at[cid, sid])
    plsc.subcore_barrier()

    @pl.when(sid == 0)
    def _():
        pltpu.sync_copy(part_hbm.at[cid], all_v)
        tot = all_v[0]
        for r in range(1, NSC):
            tot = tot + all_v[r]
        s = tot[0]
        for l in range(1, 16):
            s = s + tot[l]
        res_v[...] = jnp.broadcast_to(s * (1.0 / (B * K * C)), (16,))
        pltpu.sync_copy(res_v, res_hbm.at[cid])


_l1 = pl.kernel(
    _body,
    out_type=(
        jax.ShapeDtypeStruct((NC, NSC, 16), jnp.float32),
        jax.ShapeDtypeStruct((NC, 16), jnp.float32),
    ),
    mesh=plsc.VectorSubcoreMesh(core_axis_name="c", subcore_axis_name="s",
                                num_cores=NC),
    compiler_params=pltpu.CompilerParams(use_tc_tiling_on_sc=True,
                                         needs_layout_passes=False),
    scratch_types=[
        pltpu.VMEM((BPW, K), jnp.int32),
        pltpu.VMEM((BPW, K), jnp.int32),
        pltpu.VMEM((EPW,), jnp.float32),
        pltpu.VMEM((NCH, CHUNK), jnp.int32),
        pltpu.VMEM((NBUF, CHUNK, W), jnp.float32),
        pltpu.VMEM((16,), jnp.float32),
        pltpu.VMEM((NSC, 16), jnp.float32),
        pltpu.VMEM((16,), jnp.float32),
    ] + [pltpu.SemaphoreType.DMA] * NBUF,
)


def kernel(output, mask, ind, target):
    targ_flat = jnp.transpose(target, (0, 2, 1)).reshape(-1)
    _, res = _l1(output, ind.astype(jnp.int32), mask, targ_flat)
    return res[0, 0] + res[1, 0]


# speedup vs baseline: 2.7820x; 1.0003x over previous
import jax
import jax.numpy as jnp
from jax import lax
from jax.experimental import pallas as pl
from jax.experimental.pallas import tpu as pltpu, tpu_sc as plsc

B, C, H, W = 64, 2, 256, 256
NROWS = B * C * H
K = 128
NC = 2
NSC = 16
NWT = NC * NSC
BPW = B // NWT
EPW = BPW * K * C
CHUNK = 64
NCH = EPW // CHUNK
NBUF = 6


def _body(out_hbm, ind_hbm, mask_hbm, targ_hbm, part_hbm, res_hbm,
          ind_v, mask_v, targ_v, ridx_v, rows_v, part_v, all_v, res_v,
          *sems):
    cid = lax.axis_index("c")
    sid = lax.axis_index("s")
    wid = sid * NC + cid
    out2 = out_hbm.reshape(NROWS, W)

    pltpu.sync_copy(ind_hbm.at[pl.ds(wid * BPW, BPW)], ind_v)
    pltpu.sync_copy(mask_hbm.at[pl.ds(wid * BPW, BPW)], mask_v)
    pltpu.sync_copy(targ_hbm.at[pl.ds(wid * EPW, EPW)], targ_v)

    iota16 = lax.iota(jnp.int32, 16)

    for j in range(NCH):
        c = j >> 2
        bl = (j >> 1) & 1
        kb = (j & 1) * CHUNK
        rowbase = ((BPW * wid + bl) * C + c) * H
        for g in range(4):
            ind16 = ind_v[bl, pl.ds(kb + g * 16, 16)]
            ridx_v[j, pl.ds(g * 16, 16)] = (
                lax.shift_right_logical(ind16, 8) + rowbase)

    def fire(j):
        return pltpu.async_copy(out2.at[ridx_v.at[j]], rows_v.at[j % NBUF],
                                sems[j % NBUF])

    descs = [fire(j) for j in range(NBUF)]
    acc = jnp.zeros((16,), jnp.float32)
    for j in range(NCH):
        descs[j % NBUF].wait()
        c = j >> 2
        bl = (j >> 1) & 1
        kb = (j & 1) * CHUNK
        buf = rows_v.at[j % NBUF]
        for g in range(4):
            ind16 = ind_v[bl, pl.ds(kb + g * 16, 16)]
            col = ind16 & 255
            pred = plsc.load_gather(buf, [iota16 + g * 16, col])
            m = mask_v[bl, pl.ds(kb + g * 16, 16)].astype(jnp.float32)
            t = targ_v[pl.ds(bl * K * C + c * K + kb + g * 16, 16)]
            acc = acc + jnp.abs(pred * m - t * m)
        if j + NBUF < NCH:
            descs[j % NBUF] = fire(j + NBUF)

    part_v[...] = acc
    pltpu.sync_copy(part_v, part_hbm.at[cid, sid])
    plsc.subcore_barrier()

    @pl.when(sid == 0)
    def _():
        pltpu.sync_copy(part_hbm.at[cid], all_v)
        tot = all_v[0]
        for r in range(1, NSC):
            tot = tot + all_v[r]
        s = tot[0]
        for l in range(1, 16):
            s = s + tot[l]
        res_v[...] = jnp.broadcast_to(s * (1.0 / (B * K * C)), (16,))
        pltpu.sync_copy(res_v, res_hbm.at[cid])


_l1 = pl.kernel(
    _body,
    out_type=(
        jax.ShapeDtypeStruct((NC, NSC, 16), jnp.float32),
        jax.ShapeDtypeStruct((NC, 16), jnp.float32),
    ),
    mesh=plsc.VectorSubcoreMesh(core_axis_name="c", subcore_axis_name="s",
                                num_cores=NC),
    compiler_params=pltpu.CompilerParams(use_tc_tiling_on_sc=True,
                                         needs_layout_passes=False,
                                         skip_device_barrier=True),
    scratch_types=[
        pltpu.VMEM((BPW, K), jnp.int32),
        pltpu.VMEM((BPW, K), jnp.int32),
        pltpu.VMEM((EPW,), jnp.float32),
        pltpu.VMEM((NCH, CHUNK), jnp.int32),
        pltpu.VMEM((NBUF, CHUNK, W), jnp.float32),
        pltpu.VMEM((16,), jnp.float32),
        pltpu.VMEM((NSC, 16), jnp.float32),
        pltpu.VMEM((16,), jnp.float32),
    ] + [pltpu.SemaphoreType.DMA] * NBUF,
)


def kernel(output, mask, ind, target):
    targ_flat = jnp.transpose(target, (0, 2, 1)).reshape(-1)
    _, res = _l1(output, ind.astype(jnp.int32), mask, targ_flat)
    return res[0, 0] + res[1, 0]


# speedup vs baseline: 2.7873x; 1.0019x over previous
import jax
import jax.numpy as jnp
from jax import lax
from jax.experimental import pallas as pl
from jax.experimental.pallas import tpu as pltpu, tpu_sc as plsc

B, C, H, W = 64, 2, 256, 256
NROWS = B * C * H
K = 128
NC = 2
NSC = 16
NWT = NC * NSC
BPW = B // NWT
EPW = BPW * K * C
CHUNK = 64
NCH = EPW // CHUNK
NBUF = 6


def _body(out_hbm, ind_hbm, mask_hbm, targ_hbm, part_hbm, res_hbm,
          ind_v, mask_v, targ_v, ridx_v, rows_v, part_v, all_v, res_v,
          *sems):
    cid = lax.axis_index("c")
    sid = lax.axis_index("s")
    wid = sid * NC + cid
    out2 = out_hbm.reshape(NROWS, W)

    pltpu.sync_copy(ind_hbm.at[pl.ds(wid * BPW, BPW)], ind_v)
    pltpu.sync_copy(mask_hbm.at[pl.ds(wid * BPW, BPW)], mask_v)
    pltpu.sync_copy(targ_hbm.at[pl.ds(wid * EPW, EPW)], targ_v)

    iota16 = lax.iota(jnp.int32, 16)

    for j in range(NCH):
        c = j >> 2
        bl = (j >> 1) & 1
        kb = (j & 1) * CHUNK
        rowbase = ((BPW * wid + bl) * C + c) * H
        for g in range(4):
            ind16 = ind_v[bl, pl.ds(kb + g * 16, 16)]
            ridx_v[j, pl.ds(g * 16, 16)] = (
                lax.shift_right_logical(ind16, 8) + rowbase)

    def fire(j):
        return pltpu.async_copy(out2.at[ridx_v.at[j]], rows_v.at[j % NBUF],
                                sems[j % NBUF])

    descs = [fire(j) for j in range(NBUF)]
    acc = jnp.zeros((16,), jnp.float32)
    for j in range(NCH):
        descs[j % NBUF].wait()
        c = j >> 2
        bl = (j >> 1) & 1
        kb = (j & 1) * CHUNK
        buf = rows_v.at[j % NBUF]
        for g in range(4):
            ind16 = ind_v[bl, pl.ds(kb + g * 16, 16)]
            col = ind16 & 255
            pred = plsc.load_gather(buf, [iota16 + g * 16, col])
            m = mask_v[bl, pl.ds(kb + g * 16, 16)].astype(jnp.float32)
            t = targ_v[pl.ds(bl * K * C + c * K + kb + g * 16, 16)]
            acc = acc + jnp.abs(pred * m - t * m)
        if j + NBUF < NCH:
            descs[j % NBUF] = fire(j + NBUF)

    part_v[...] = acc
    pltpu.sync_copy(part_v, part_hbm.at[cid, sid])
    plsc.subcore_barrier()

    @pl.when(sid == 0)
    def _():
        pltpu.sync_copy(part_hbm.at[cid], all_v)
        tot = all_v[0]
        for r in range(1, NSC):
            tot = tot + all_v[r]
        s = tot[0]
        for l in range(1, 16):
            s = s + tot[l]
        res_v[...] = jnp.broadcast_to(s * (1.0 / (B * K * C)), (16,))
        pltpu.sync_copy(res_v, res_hbm.at[cid])


_l1 = pl.kernel(
    _body,
    out_type=(
        jax.ShapeDtypeStruct((NC, NSC, 16), jnp.float32),
        jax.ShapeDtypeStruct((NC, 16), jnp.float32),
    ),
    mesh=plsc.VectorSubcoreMesh(core_axis_name="c", subcore_axis_name="s",
                                num_cores=NC),
    compiler_params=pltpu.CompilerParams(use_tc_tiling_on_sc=True,
                                         needs_layout_passes=False),
    scratch_types=[
        pltpu.VMEM((BPW, K), jnp.int32),
        pltpu.VMEM((BPW, K), jnp.int32),
        pltpu.VMEM((EPW,), jnp.float32),
        pltpu.VMEM((NCH, CHUNK), jnp.int32),
        pltpu.VMEM((NBUF, CHUNK, W), jnp.float32),
        pltpu.VMEM((16,), jnp.float32),
        pltpu.VMEM((NSC, 16), jnp.float32),
        pltpu.VMEM((16,), jnp.float32),
    ] + [pltpu.SemaphoreType.DMA] * NBUF,
)


def kernel(output, mask, ind, target):
    targ_flat = jnp.transpose(target, (0, 2, 1)).reshape(-1)
    _, res = _l1(output, ind.astype(jnp.int32), mask, targ_flat)
    return res[0, 0] + res[1, 0]
